# baseline (speedup 1.0000x reference)
import functools

import jax
import jax.numpy as jnp
from jax.experimental import pallas as pl
from jax.experimental.pallas import tpu as pltpu

CHEB_K = 3
TOP_K = 2


def _fused_body(x_ref, emb_w_ref, emb_b_ref, pos_ref, sup_ref, gate_w_ref,
                exp_w_ref, exp_b_ref, o_ref, h_ref, out_ref,
                xt_ref, a_ref, t2_ref, *, T, TB, n_sup, n_exp):
    f32 = jnp.float32
    b = pl.program_id(0)
    tb = pl.program_id(1)
    n_tb = T // TB

    @pl.when(tb < n_tb)
    def _embed():
        wr = emb_w_ref[...].astype(f32)
        C = wr.shape[0]
        acc = None
        for i in range(TB):
            xb = jnp.transpose(x_ref[0, i]).astype(f32)
            bias = emb_b_ref[0:1, :] + pos_ref[i]
            val = xb[:, 0:1] * wr[0:1, :] + bias
            for c in range(1, C):
                val = val + xb[:, c : c + 1] * wr[c : c + 1, :]
            out_ref[0, i] = val
            acc = val if acc is None else acc + val

        @pl.when(tb == 0)
        def _init():
            xt_ref[...] = acc

        @pl.when(tb > 0)
        def _acc():
            xt_ref[...] += acc

    @pl.when((b == 0) & (tb == 0))
    def _build_t2():
        N = sup_ref.shape[1]
        row = jax.lax.broadcasted_iota(jnp.int32, (N, N), 0)
        col = jax.lax.broadcasted_iota(jnp.int32, (N, N), 1)
        eye = (row == col).astype(f32)
        for s in range(n_sup):
            a_bf = sup_ref[s].astype(jnp.bfloat16)
            a_ref[s] = a_bf
            t2 = 2.0 * jnp.dot(a_bf, a_bf, preferred_element_type=f32) - eye
            t2_ref[s] = t2.astype(jnp.bfloat16)

    @pl.when(tb == n_tb)
    def _expert():
        bf = jnp.bfloat16
        xt = xt_ref[...] / float(T)
        N = xt.shape[0]
        xt_bf = xt.astype(bf)

        chunks = []
        for s in range(n_sup):
            z1 = jnp.dot(a_ref[s], xt_bf, preferred_element_type=f32)
            z2 = jnp.dot(t2_ref[s], xt_bf, preferred_element_type=f32)
            chunks.extend([xt_bf, z1.astype(bf), z2.astype(bf)])
        xg = jnp.concatenate(chunks, axis=1)

        gate = jnp.dot(xt_bf, gate_w_ref[...].astype(bf),
                       preferred_element_type=f32)
        iota = jax.lax.broadcasted_iota(jnp.int32, (N, n_exp), 1)
        m1 = jnp.max(gate, axis=1, keepdims=True)
        idx1 = jnp.min(jnp.where(gate == m1, iota, n_exp), axis=1, keepdims=True)
        masked = jnp.where(iota == idx1, -jnp.inf, gate)
        m2 = jnp.max(masked, axis=1, keepdims=True)
        idx2 = jnp.min(jnp.where(masked == m2, iota, n_exp), axis=1, keepdims=True)
        e1 = jnp.exp(m2 - m1)
        denom = 1.0 + e1
        w1 = 1.0 / denom
        w2 = e1 / denom

        o = jnp.zeros_like(xt)
        for e in range(n_exp):
            oe = jnp.dot(xg, exp_w_ref[e].astype(bf), preferred_element_type=f32)
            oe = oe + exp_b_ref[e : e + 1, :]
            coef = jnp.where(idx1 == e, w1, 0.0) + jnp.where(idx2 == e, w2, 0.0)
            o = o + coef * oe
        o_ref[0] = o
        h_ref[0] = jnp.tanh(o)


@jax.jit
def kernel(x, y_cov, supports, emb_W, emb_b, pos_emb, gate_W, exp_W, exp_b):
    B, T, N, C = x.shape
    D = emb_W.shape[1]
    n_sup = supports.shape[0]
    n_exp = exp_W.shape[0]
    TB = 12
    n_tb = T // TB
    last = n_tb - 1

    o_expert, h_expert, current_inputs = pl.pallas_call(
        functools.partial(_fused_body, T=T, TB=TB, n_sup=n_sup, n_exp=n_exp),
        grid=(B, n_tb + 1),
        in_specs=[
            pl.BlockSpec((1, TB, C, N), lambda b, t: (b, jnp.minimum(t, last), 0, 0)),
            pl.BlockSpec((C, D), lambda b, t: (0, 0)),
            pl.BlockSpec((1, D), lambda b, t: (0, 0)),
            pl.BlockSpec((TB, 1, D), lambda b, t: (jnp.minimum(t, last), 0, 0)),
            pl.BlockSpec((n_sup, N, N), lambda b, t: (0, 0, 0)),
            pl.BlockSpec((D, n_exp), lambda b, t: (0, 0)),
            pl.BlockSpec((n_exp, 2 * CHEB_K * D, D), lambda b, t: (0, 0, 0)),
            pl.BlockSpec((n_exp, D), lambda b, t: (0, 0)),
        ],
        out_specs=[
            pl.BlockSpec((1, N, D), lambda b, t: (b, 0, 0)),
            pl.BlockSpec((1, N, D), lambda b, t: (b, 0, 0)),
            pl.BlockSpec((1, TB, N, D), lambda b, t: (b, jnp.minimum(t, last), 0, 0)),
        ],
        out_shape=[
            jax.ShapeDtypeStruct((B, N, D), jnp.float32),
            jax.ShapeDtypeStruct((B, N, D), jnp.float32),
            jax.ShapeDtypeStruct((B, T, N, D), jnp.float32),
        ],
        scratch_shapes=[
            pltpu.VMEM((N, D), jnp.float32),
            pltpu.VMEM((n_sup, N, N), jnp.bfloat16),
            pltpu.VMEM((n_sup, N, N), jnp.bfloat16),
        ],
    )(jnp.swapaxes(x, 2, 3).astype(jnp.bfloat16), emb_W.astype(jnp.bfloat16),
      emb_b.reshape(1, D), pos_emb.reshape(T, 1, D), supports, gate_W, exp_W,
      exp_b)

    return (o_expert, h_expert, current_inputs)


# speedup vs baseline: 1.0435x; 1.0435x over previous
import functools

import jax
import jax.numpy as jnp
from jax.experimental import pallas as pl
from jax.experimental.pallas import tpu as pltpu

CHEB_K = 3
TOP_K = 2


def _fused_body(x_ref, emb_w_ref, emb_b_ref, pos_ref, sup_ref, gate_w_ref,
                exp_w_ref, exp_b_ref, o_ref, h_ref, out_ref,
                xt_ref, a_ref, t2_ref, *, T, TB, n_sup, n_exp):
    f32 = jnp.float32
    b = pl.program_id(0)
    tb = pl.program_id(1)
    n_tb = T // TB

    @pl.when(tb < n_tb)
    def _embed():
        wr = emb_w_ref[...].astype(jnp.bfloat16).astype(f32)
        C = wr.shape[0]
        acc = None
        for i in range(TB):
            xb = jnp.transpose(x_ref[0, i]).astype(jnp.bfloat16).astype(f32)
            bias = emb_b_ref[0:1, :] + pos_ref[i]
            val = xb[:, 0:1] * wr[0:1, :] + bias
            for c in range(1, C):
                val = val + xb[:, c : c + 1] * wr[c : c + 1, :]
            out_ref[0, i] = val
            acc = val if acc is None else acc + val

        @pl.when(tb == 0)
        def _init():
            xt_ref[...] = acc

        @pl.when(tb > 0)
        def _acc():
            xt_ref[...] += acc

    @pl.when((b == 0) & (tb == 0))
    def _build_t2():
        N = sup_ref.shape[1]
        row = jax.lax.broadcasted_iota(jnp.int32, (N, N), 0)
        col = jax.lax.broadcasted_iota(jnp.int32, (N, N), 1)
        eye = (row == col).astype(f32)
        for s in range(n_sup):
            a_bf = sup_ref[s].astype(jnp.bfloat16)
            a_ref[s] = a_bf
            t2 = 2.0 * jnp.dot(a_bf, a_bf, preferred_element_type=f32) - eye
            t2_ref[s] = t2.astype(jnp.bfloat16)

    @pl.when(tb == n_tb)
    def _expert():
        bf = jnp.bfloat16
        xt = xt_ref[...] / float(T)
        N = xt.shape[0]
        xt_bf = xt.astype(bf)

        chunks = []
        for s in range(n_sup):
            z1 = jnp.dot(a_ref[s], xt_bf, preferred_element_type=f32)
            z2 = jnp.dot(t2_ref[s], xt_bf, preferred_element_type=f32)
            chunks.extend([xt_bf, z1.astype(bf), z2.astype(bf)])
        xg = jnp.concatenate(chunks, axis=1)

        gate = jnp.dot(xt_bf, gate_w_ref[...].astype(bf),
                       preferred_element_type=f32)
        iota = jax.lax.broadcasted_iota(jnp.int32, (N, n_exp), 1)
        m1 = jnp.max(gate, axis=1, keepdims=True)
        idx1 = jnp.min(jnp.where(gate == m1, iota, n_exp), axis=1, keepdims=True)
        masked = jnp.where(iota == idx1, -jnp.inf, gate)
        m2 = jnp.max(masked, axis=1, keepdims=True)
        idx2 = jnp.min(jnp.where(masked == m2, iota, n_exp), axis=1, keepdims=True)
        e1 = jnp.exp(m2 - m1)
        denom = 1.0 + e1
        w1 = 1.0 / denom
        w2 = e1 / denom

        o = jnp.zeros_like(xt)
        for e in range(n_exp):
            oe = jnp.dot(xg, exp_w_ref[e].astype(bf), preferred_element_type=f32)
            oe = oe + exp_b_ref[e : e + 1, :]
            coef = jnp.where(idx1 == e, w1, 0.0) + jnp.where(idx2 == e, w2, 0.0)
            o = o + coef * oe
        o_ref[0] = o
        h_ref[0] = jnp.tanh(o)


@jax.jit
def kernel(x, y_cov, supports, emb_W, emb_b, pos_emb, gate_W, exp_W, exp_b):
    B, T, N, C = x.shape
    D = emb_W.shape[1]
    n_sup = supports.shape[0]
    n_exp = exp_W.shape[0]
    TB = 24
    n_tb = T // TB
    last = n_tb - 1

    o_expert, h_expert, current_inputs = pl.pallas_call(
        functools.partial(_fused_body, T=T, TB=TB, n_sup=n_sup, n_exp=n_exp),
        grid=(B, n_tb + 1),
        in_specs=[
            pl.BlockSpec((1, TB, C, N), lambda b, t: (b, jnp.minimum(t, last), 0, 0)),
            pl.BlockSpec((C, D), lambda b, t: (0, 0)),
            pl.BlockSpec((1, D), lambda b, t: (0, 0)),
            pl.BlockSpec((TB, 1, D), lambda b, t: (jnp.minimum(t, last), 0, 0)),
            pl.BlockSpec((n_sup, N, N), lambda b, t: (0, 0, 0)),
            pl.BlockSpec((D, n_exp), lambda b, t: (0, 0)),
            pl.BlockSpec((n_exp, 2 * CHEB_K * D, D), lambda b, t: (0, 0, 0)),
            pl.BlockSpec((n_exp, D), lambda b, t: (0, 0)),
        ],
        out_specs=[
            pl.BlockSpec((1, N, D), lambda b, t: (b, 0, 0)),
            pl.BlockSpec((1, N, D), lambda b, t: (b, 0, 0)),
            pl.BlockSpec((1, TB, N, D), lambda b, t: (b, jnp.minimum(t, last), 0, 0)),
        ],
        out_shape=[
            jax.ShapeDtypeStruct((B, N, D), jnp.float32),
            jax.ShapeDtypeStruct((B, N, D), jnp.float32),
            jax.ShapeDtypeStruct((B, T, N, D), jnp.float32),
        ],
        scratch_shapes=[
            pltpu.VMEM((N, D), jnp.float32),
            pltpu.VMEM((n_sup, N, N), jnp.bfloat16),
            pltpu.VMEM((n_sup, N, N), jnp.bfloat16),
        ],
    )(jnp.swapaxes(x, 2, 3), emb_W, emb_b.reshape(1, D),
      pos_emb.reshape(T, 1, D), supports, gate_W, exp_W, exp_b)

    return (o_expert, h_expert, current_inputs)


# speedup vs baseline: 1.0608x; 1.0166x over previous
import functools

import jax
import jax.numpy as jnp
from jax.experimental import pallas as pl
from jax.experimental.pallas import tpu as pltpu

CHEB_K = 3
TOP_K = 2


def _fused_body(x_ref, emb_w_ref, emb_b_ref, pos_ref, sup_ref, gate_w_ref,
                exp_w_ref, exp_b_ref, o_ref, h_ref, out_ref,
                xt_ref, a_ref, t2_ref, *, T, TB, n_sup, n_exp):
    f32 = jnp.float32
    b = pl.program_id(0)
    tb = pl.program_id(1)
    n_tb = T // TB

    @pl.when(tb < n_tb)
    def _embed():
        wr = emb_w_ref[...].astype(jnp.bfloat16).astype(f32)
        C = wr.shape[0]
        xT = x_ref[0].astype(jnp.bfloat16).astype(f32)
        acc = None
        for i in range(TB):
            t = i
            bias = emb_b_ref[0:1, :] + pos_ref[i]
            val = xT[:, t * C : t * C + 1] * wr[0:1, :] + bias
            for c in range(1, C):
                val = val + xT[:, t * C + c : t * C + c + 1] * wr[c : c + 1, :]
            out_ref[0, i] = val
            acc = val if acc is None else acc + val

        @pl.when(tb == 0)
        def _init():
            xt_ref[...] = acc

        @pl.when(tb > 0)
        def _acc():
            xt_ref[...] += acc

    @pl.when((b == 0) & (tb == 0))
    def _build_t2():
        N = sup_ref.shape[1]
        row = jax.lax.broadcasted_iota(jnp.int32, (N, N), 0)
        col = jax.lax.broadcasted_iota(jnp.int32, (N, N), 1)
        eye = (row == col).astype(f32)
        for s in range(n_sup):
            a_bf = sup_ref[s].astype(jnp.bfloat16)
            a_ref[s] = a_bf
            t2 = 2.0 * jnp.dot(a_bf, a_bf, preferred_element_type=f32) - eye
            t2_ref[s] = t2.astype(jnp.bfloat16)

    @pl.when(tb == n_tb)
    def _expert():
        bf = jnp.bfloat16
        xt = xt_ref[...] / float(T)
        N = xt.shape[0]
        xt_bf = xt.astype(bf)

        chunks = []
        for s in range(n_sup):
            z1 = jnp.dot(a_ref[s], xt_bf, preferred_element_type=f32)
            z2 = jnp.dot(t2_ref[s], xt_bf, preferred_element_type=f32)
            chunks.extend([xt_bf, z1.astype(bf), z2.astype(bf)])
        xg = jnp.concatenate(chunks, axis=1)

        gate = jnp.dot(xt_bf, gate_w_ref[...].astype(bf),
                       preferred_element_type=f32)
        iota = jax.lax.broadcasted_iota(jnp.int32, (N, n_exp), 1)
        m1 = jnp.max(gate, axis=1, keepdims=True)
        idx1 = jnp.min(jnp.where(gate == m1, iota, n_exp), axis=1, keepdims=True)
        masked = jnp.where(iota == idx1, -jnp.inf, gate)
        m2 = jnp.max(masked, axis=1, keepdims=True)
        idx2 = jnp.min(jnp.where(masked == m2, iota, n_exp), axis=1, keepdims=True)
        e1 = jnp.exp(m2 - m1)
        denom = 1.0 + e1
        w1 = 1.0 / denom
        w2 = e1 / denom

        o = jnp.zeros_like(xt)
        for e in range(n_exp):
            oe = jnp.dot(xg, exp_w_ref[e].astype(bf), preferred_element_type=f32)
            oe = oe + exp_b_ref[e : e + 1, :]
            coef = jnp.where(idx1 == e, w1, 0.0) + jnp.where(idx2 == e, w2, 0.0)
            o = o + coef * oe
        o_ref[0] = o
        h_ref[0] = jnp.tanh(o)


@jax.jit
def kernel(x, y_cov, supports, emb_W, emb_b, pos_emb, gate_W, exp_W, exp_b):
    B, T, N, C = x.shape
    D = emb_W.shape[1]
    n_sup = supports.shape[0]
    n_exp = exp_W.shape[0]
    TB = 24
    n_tb = T // TB
    last = n_tb - 1

    o_expert, h_expert, current_inputs = pl.pallas_call(
        functools.partial(_fused_body, T=T, TB=TB, n_sup=n_sup, n_exp=n_exp),
        grid=(B, n_tb + 1),
        in_specs=[
            pl.BlockSpec((1, N, T * C), lambda b, t: (b, 0, 0)),
            pl.BlockSpec((C, D), lambda b, t: (0, 0)),
            pl.BlockSpec((1, D), lambda b, t: (0, 0)),
            pl.BlockSpec((TB, 1, D), lambda b, t: (jnp.minimum(t, last), 0, 0)),
            pl.BlockSpec((n_sup, N, N), lambda b, t: (0, 0, 0)),
            pl.BlockSpec((D, n_exp), lambda b, t: (0, 0)),
            pl.BlockSpec((n_exp, 2 * CHEB_K * D, D), lambda b, t: (0, 0, 0)),
            pl.BlockSpec((n_exp, D), lambda b, t: (0, 0)),
        ],
        out_specs=[
            pl.BlockSpec((1, N, D), lambda b, t: (b, 0, 0)),
            pl.BlockSpec((1, N, D), lambda b, t: (b, 0, 0)),
            pl.BlockSpec((1, TB, N, D), lambda b, t: (b, jnp.minimum(t, last), 0, 0)),
        ],
        out_shape=[
            jax.ShapeDtypeStruct((B, N, D), jnp.float32),
            jax.ShapeDtypeStruct((B, N, D), jnp.float32),
            jax.ShapeDtypeStruct((B, T, N, D), jnp.float32),
        ],
        scratch_shapes=[
            pltpu.VMEM((N, D), jnp.float32),
            pltpu.VMEM((n_sup, N, N), jnp.bfloat16),
            pltpu.VMEM((n_sup, N, N), jnp.bfloat16),
        ],
    )(x.transpose(0, 2, 1, 3).reshape(B, N, T * C), emb_W, emb_b.reshape(1, D),
      pos_emb.reshape(T, 1, D), supports, gate_W, exp_W, exp_b)

    return (o_expert, h_expert, current_inputs)


# speedup vs baseline: 1.1585x; 1.0921x over previous
import functools

import jax
import jax.numpy as jnp
from jax.experimental import pallas as pl
from jax.experimental.pallas import tpu as pltpu

CHEB_K = 3
TOP_K = 2


def _fused_body(x_ref, emb_w_ref, emb_b_ref, pos_ref, a_ref, gate_w_ref,
                wall_ref, exp_b_ref, o_ref, h_ref, out_ref,
                xt_ref, t2_ref, *, B, T, n_sup, n_exp):
    f32 = jnp.float32
    bf = jnp.bfloat16
    g = pl.program_id(0)

    @pl.when(g < B)
    def _embed():
        wr = emb_w_ref[...].astype(bf).astype(f32)
        C = wr.shape[0]
        D = wr.shape[1]
        xT = x_ref[0].astype(bf).astype(f32)
        acc = None
        for t in range(T):
            bias = emb_b_ref[0:1, :] + pos_ref[t]
            val = xT[:, t * C : t * C + 1] * wr[0:1, :] + bias
            for c in range(1, C):
                val = val + xT[:, t * C + c : t * C + c + 1] * wr[c : c + 1, :]
            out_ref[0, t] = val
            acc = val if acc is None else acc + val
        xt_ref[:, pl.ds(g * D, D)] = acc

    @pl.when(g == 0)
    def _build_t2():
        N = a_ref.shape[1]
        row = jax.lax.broadcasted_iota(jnp.int32, (N, N), 0)
        col = jax.lax.broadcasted_iota(jnp.int32, (N, N), 1)
        eye = (row == col).astype(f32)
        for s in range(n_sup):
            t2 = 2.0 * jnp.dot(a_ref[s], a_ref[s], preferred_element_type=f32) - eye
            t2_ref[s] = t2.astype(bf)

    @pl.when(g == B)
    def _expert():
        N = xt_ref.shape[0]
        D = emb_w_ref.shape[1]
        xt_bf = (xt_ref[...] / float(T)).astype(bf)

        zs = []
        for s in range(n_sup):
            z1 = jnp.dot(a_ref[s], xt_bf, preferred_element_type=f32)
            z2 = jnp.dot(t2_ref[s], xt_bf, preferred_element_type=f32)
            zs.append((z1.astype(bf), z2.astype(bf)))

        iota = jax.lax.broadcasted_iota(jnp.int32, (N, n_exp), 1)
        for b in range(B):
            sl = slice(b * D, (b + 1) * D)
            xt_b = xt_bf[:, sl]
            chunks = []
            for s in range(n_sup):
                chunks.extend([xt_b, zs[s][0][:, sl], zs[s][1][:, sl]])
            xg = jnp.concatenate(chunks, axis=1)

            gate = jnp.dot(xt_b, gate_w_ref[...], preferred_element_type=f32)
            m1 = jnp.max(gate, axis=1, keepdims=True)
            idx1 = jnp.min(jnp.where(gate == m1, iota, n_exp), axis=1,
                           keepdims=True)
            masked = jnp.where(iota == idx1, -jnp.inf, gate)
            m2 = jnp.max(masked, axis=1, keepdims=True)
            idx2 = jnp.min(jnp.where(masked == m2, iota, n_exp), axis=1,
                           keepdims=True)
            e1 = jnp.exp(m2 - m1)
            denom = 1.0 + e1
            w1 = 1.0 / denom
            w2 = e1 / denom

            oe_all = jnp.dot(xg, wall_ref[...], preferred_element_type=f32)
            o = jnp.zeros((N, D), f32)
            for e in range(n_exp):
                oe = oe_all[:, e * D : (e + 1) * D] + exp_b_ref[e : e + 1, :]
                coef = (jnp.where(idx1 == e, w1, 0.0)
                        + jnp.where(idx2 == e, w2, 0.0))
                o = o + coef * oe
            o_ref[b] = o
            h_ref[b] = jnp.tanh(o)


@jax.jit
def kernel(x, y_cov, supports, emb_W, emb_b, pos_emb, gate_W, exp_W, exp_b):
    B, T, N, C = x.shape
    D = emb_W.shape[1]
    n_sup = supports.shape[0]
    n_exp = exp_W.shape[0]
    bf = jnp.bfloat16

    o_expert, h_expert, current_inputs = pl.pallas_call(
        functools.partial(_fused_body, B=B, T=T, n_sup=n_sup, n_exp=n_exp),
        grid=(B + 1,),
        in_specs=[
            pl.BlockSpec((1, N, T * C), lambda g: (jnp.minimum(g, B - 1), 0, 0)),
            pl.BlockSpec((C, D), lambda g: (0, 0)),
            pl.BlockSpec((1, D), lambda g: (0, 0)),
            pl.BlockSpec((T, 1, D), lambda g: (0, 0, 0)),
            pl.BlockSpec((n_sup, N, N), lambda g: (0, 0, 0)),
            pl.BlockSpec((D, n_exp), lambda g: (0, 0)),
            pl.BlockSpec((2 * CHEB_K * D, n_exp * D), lambda g: (0, 0)),
            pl.BlockSpec((n_exp, D), lambda g: (0, 0)),
        ],
        out_specs=[
            pl.BlockSpec((B, N, D), lambda g: (0, 0, 0)),
            pl.BlockSpec((B, N, D), lambda g: (0, 0, 0)),
            pl.BlockSpec((1, T, N, D), lambda g: (jnp.minimum(g, B - 1), 0, 0, 0)),
        ],
        out_shape=[
            jax.ShapeDtypeStruct((B, N, D), jnp.float32),
            jax.ShapeDtypeStruct((B, N, D), jnp.float32),
            jax.ShapeDtypeStruct((B, T, N, D), jnp.float32),
        ],
        scratch_shapes=[
            pltpu.VMEM((N, B * D), jnp.float32),
            pltpu.VMEM((n_sup, N, N), jnp.bfloat16),
        ],
    )(x.transpose(0, 2, 1, 3).reshape(B, N, T * C), emb_W,
      emb_b.reshape(1, D), pos_emb.reshape(T, 1, D), supports.astype(bf),
      gate_W.astype(bf), exp_W.transpose(1, 0, 2).reshape(2 * CHEB_K * D,
                                                          n_exp * D).astype(bf),
      exp_b)

    return (o_expert, h_expert, current_inputs)
